# baseline (speedup 1.0000x reference)
import jax
import jax.numpy as jnp
import numpy as np
from jax import lax
from jax.experimental import pallas as pl

_N = 4096
_D = 128
_K = 20
_BM = 256
_DOM = 128


def _rotl(x, d):
    return ((x << np.uint32(d)) | (x >> np.uint32(32 - d))).astype(np.uint32)


def _threefry2x32(k0, k1, x0, x1):
    rot = ((13, 15, 26, 6), (17, 29, 16, 24))
    ks = (np.uint32(k0), np.uint32(k1),
          np.uint32(np.uint32(k0) ^ np.uint32(k1) ^ np.uint32(0x1BD11BDA)))
    x0 = (x0 + ks[0]).astype(np.uint32)
    x1 = (x1 + ks[1]).astype(np.uint32)
    for i in range(5):
        for r in rot[i % 2]:
            x0 = (x0 + x1).astype(np.uint32)
            x1 = _rotl(x1, r)
            x1 = (x0 ^ x1).astype(np.uint32)
        x0 = (x0 + ks[(i + 1) % 3]).astype(np.uint32)
        x1 = (x1 + ks[(i + 2) % 3] + np.uint32(i + 1)).astype(np.uint32)
    return x0, x1


def _rank_tables():
    n = _N * _N
    b0, b1 = _threefry2x32(0, 42, np.zeros(n, np.uint32),
                           np.arange(n, dtype=np.uint32))
    bits = b0 ^ b1
    u = ((bits >> np.uint32(9)) | np.uint32(0x3F800000)).view(np.float32)
    scores = np.maximum(np.float32(0.0), u - np.float32(1.0)).reshape(_N, _N)
    order = np.argsort(-scores, axis=1, kind="stable")
    ranks = np.argsort(order, axis=1, kind="stable")
    return ranks.astype(np.int16), np.minimum(ranks, _DOM - 1).astype(np.uint8)


_RANKS16, _RANKS8 = _rank_tables()


def _finish(s, ls, af, inv_cnt, sel, o_ref):
    inv_k = jnp.float32(1.0 / _K)
    w = af * inv_cnt + sel * inv_k
    term_ls = jnp.sum(w * ls, axis=1, keepdims=True)
    term_s = jnp.sum(sel * s, axis=1, keepdims=True) * inv_k
    o_ref[...] = jnp.reshape(jnp.sum(term_s - term_ls), (1, 1, 1))


def _fast_body(x_ref, xf_ref, a_ref, r8_ref, o_ref, f_ref):
    x = x_ref[...]
    xf = xf_ref[...]
    s = lax.dot_general(x, xf, (((1,), (1,)), ((), ())),
                        preferred_element_type=jnp.float32)

    af = a_ref[...].astype(jnp.float32)
    ls = jnp.minimum(s, 0.0) - jnp.log1p(jnp.exp(-jnp.abs(s)))

    pos_cnt = jnp.sum(af, axis=1, keepdims=True)
    inv_cnt = 1.0 / jnp.maximum(pos_cnt, 1.0)

    m8 = r8_ref[...].astype(jnp.float32) + af * jnp.float32(_DOM)
    lo = jnp.zeros((_BM, 1), jnp.int32)
    hi = jnp.full((_BM, 1), _DOM - 1, jnp.int32)
    kf = jnp.float32(_K)
    for _ in range(7):
        mid = (lo + hi) >> 1
        midf = mid.astype(jnp.float32)
        cnt = jnp.sum(jnp.where(m8 <= midf, 1.0, 0.0), axis=1, keepdims=True)
        ge = cnt >= kf
        hi = jnp.where(ge, mid, hi)
        lo = jnp.where(ge, lo, mid + 1)
    bad = lo >= _DOM - 1
    f_ref[...] = jnp.reshape(jnp.sum(bad.astype(jnp.float32)), (1, 1, 1))

    sel = jnp.where(m8 <= lo.astype(jnp.float32), 1.0, 0.0)
    _finish(s, ls, af, inv_cnt, sel, o_ref)


def _full_body(x_ref, xf_ref, a_ref, r16_ref, o_ref):
    x = x_ref[...]
    xf = xf_ref[...]
    s = lax.dot_general(x, xf, (((1,), (1,)), ((), ())),
                        preferred_element_type=jnp.float32)

    a = a_ref[...]
    af = a.astype(jnp.float32)
    ls = jnp.minimum(s, 0.0) - jnp.log1p(jnp.exp(-jnp.abs(s)))

    pos_cnt = jnp.sum(af, axis=1, keepdims=True)
    inv_cnt = 1.0 / jnp.maximum(pos_cnt, 1.0)

    mrank = r16_ref[...].astype(jnp.int32) + (a << 12)
    lo = jnp.zeros((_BM, 1), jnp.int32)
    hi = jnp.full((_BM, 1), _N - 1, jnp.int32)
    for _ in range(12):
        mid = (lo + hi) >> 1
        cnt = jnp.sum((mrank <= mid).astype(jnp.float32),
                      axis=1, keepdims=True)
        ge = cnt >= _K
        hi = jnp.where(ge, mid, hi)
        lo = jnp.where(ge, lo, mid + 1)
    sel = jnp.where(mrank <= lo, 1.0, 0.0)
    _finish(s, ls, af, inv_cnt, sel, o_ref)


def kernel(X, A):
    X2 = X[0]
    A2 = A[0].astype(jnp.int32)
    grid = _N // _BM
    blk = lambda i: (i, 0)
    out_spec = pl.BlockSpec((1, 1, 1), lambda i: (i, 0, 0))
    out_shape = jax.ShapeDtypeStruct((grid, 1, 1), jnp.float32)

    partials, flags = pl.pallas_call(
        _fast_body,
        grid=(grid,),
        in_specs=[
            pl.BlockSpec((_BM, _D), blk),
            pl.BlockSpec((_N, _D), lambda i: (0, 0)),
            pl.BlockSpec((_BM, _N), blk),
            pl.BlockSpec((_BM, _N), blk),
        ],
        out_specs=(out_spec, out_spec),
        out_shape=(out_shape, out_shape),
    )(X2, X2, A2, jnp.asarray(_RANKS8))

    def full_loss():
        parts = pl.pallas_call(
            _full_body,
            grid=(grid,),
            in_specs=[
                pl.BlockSpec((_BM, _D), blk),
                pl.BlockSpec((_N, _D), lambda i: (0, 0)),
                pl.BlockSpec((_BM, _N), blk),
                pl.BlockSpec((_BM, _N), blk),
            ],
            out_specs=out_spec,
            out_shape=out_shape,
        )(X2, X2, A2, jnp.asarray(_RANKS16))
        return jnp.sum(parts)

    return lax.cond(jnp.sum(flags) > 0.0, full_loss,
                    lambda: jnp.sum(partials))


# speedup vs baseline: 1.1371x; 1.1371x over previous
import jax
import jax.numpy as jnp
import numpy as np
from jax import lax
from jax.experimental import pallas as pl

_N = 4096
_D = 128
_K = 20
_BM = 256
_DOM = 128


def _rotl(x, d):
    return ((x << np.uint32(d)) | (x >> np.uint32(32 - d))).astype(np.uint32)


def _threefry2x32(k0, k1, x0, x1):
    rot = ((13, 15, 26, 6), (17, 29, 16, 24))
    ks = (np.uint32(k0), np.uint32(k1),
          np.uint32(np.uint32(k0) ^ np.uint32(k1) ^ np.uint32(0x1BD11BDA)))
    x0 = (x0 + ks[0]).astype(np.uint32)
    x1 = (x1 + ks[1]).astype(np.uint32)
    for i in range(5):
        for r in rot[i % 2]:
            x0 = (x0 + x1).astype(np.uint32)
            x1 = _rotl(x1, r)
            x1 = (x0 ^ x1).astype(np.uint32)
        x0 = (x0 + ks[(i + 1) % 3]).astype(np.uint32)
        x1 = (x1 + ks[(i + 2) % 3] + np.uint32(i + 1)).astype(np.uint32)
    return x0, x1


def _rank_tables():
    n = _N * _N
    b0, b1 = _threefry2x32(0, 42, np.zeros(n, np.uint32),
                           np.arange(n, dtype=np.uint32))
    bits = b0 ^ b1
    u = ((bits >> np.uint32(9)) | np.uint32(0x3F800000)).view(np.float32)
    scores = np.maximum(np.float32(0.0), u - np.float32(1.0)).reshape(_N, _N)
    order = np.argsort(-scores, axis=1, kind="stable")
    ranks = np.argsort(order, axis=1, kind="stable")
    return ranks.astype(np.int16), np.minimum(ranks, _DOM - 1).astype(np.uint8)


_RANKS16, _RANKS8 = _rank_tables()


def _mxu_rowsum(v, ones):
    r = lax.dot_general(v, ones, (((1,), (0,)), ((), ())),
                        preferred_element_type=jnp.float32)
    return r[:, :1]


def _finish(s, ls, af, inv_cnt, sel, ones, o_ref):
    inv_k = jnp.float32(1.0 / _K)
    w = af * inv_cnt + sel * inv_k
    term_ls = _mxu_rowsum(w * ls, ones)
    term_s = _mxu_rowsum(sel * s, ones) * inv_k
    o_ref[...] += jnp.reshape(jnp.sum(term_s - term_ls), (1, 1, 1))


def _fast_body(x_ref, xf_ref, a_ref, r8_ref, o_ref, f_ref):
    @pl.when(pl.program_id(0) == 0)
    def _():
        o_ref[...] = jnp.zeros_like(o_ref)
        f_ref[...] = jnp.zeros_like(f_ref)

    x = x_ref[...]
    xf = xf_ref[...]
    s = lax.dot_general(x, xf, (((1,), (1,)), ((), ())),
                        preferred_element_type=jnp.float32)

    af = a_ref[...].astype(jnp.float32)
    ones = jnp.ones((_N, 128), jnp.float32)
    ls = jnp.minimum(s, 0.0) - jnp.log1p(jnp.exp(-jnp.abs(s)))

    pos_cnt = _mxu_rowsum(af, ones)
    inv_cnt = 1.0 / jnp.maximum(pos_cnt, 1.0)

    m8 = r8_ref[...].astype(jnp.float32) + af * jnp.float32(_DOM)
    lo = jnp.zeros((_BM, 1), jnp.int32)
    hi = jnp.full((_BM, 1), _DOM - 1, jnp.int32)
    kf = jnp.float32(_K)
    for _ in range(7):
        mid = (lo + hi) >> 1
        midf = mid.astype(jnp.float32)
        cnt = jnp.sum(jnp.where(m8 <= midf, 1.0, 0.0), axis=1, keepdims=True)
        ge = cnt >= kf
        hi = jnp.where(ge, mid, hi)
        lo = jnp.where(ge, lo, mid + 1)
    bad = lo >= _DOM - 1
    f_ref[...] += jnp.reshape(jnp.sum(bad.astype(jnp.float32)), (1, 1, 1))

    sel = jnp.where(m8 <= lo.astype(jnp.float32), 1.0, 0.0)
    _finish(s, ls, af, inv_cnt, sel, ones, o_ref)


def _full_body(x_ref, xf_ref, a_ref, r16_ref, o_ref):
    @pl.when(pl.program_id(0) == 0)
    def _():
        o_ref[...] = jnp.zeros_like(o_ref)

    x = x_ref[...]
    xf = xf_ref[...]
    s = lax.dot_general(x, xf, (((1,), (1,)), ((), ())),
                        preferred_element_type=jnp.float32)

    a = a_ref[...]
    af = a.astype(jnp.float32)
    ones = jnp.ones((_N, 128), jnp.float32)
    ls = jnp.minimum(s, 0.0) - jnp.log1p(jnp.exp(-jnp.abs(s)))

    pos_cnt = _mxu_rowsum(af, ones)
    inv_cnt = 1.0 / jnp.maximum(pos_cnt, 1.0)

    mrank = r16_ref[...].astype(jnp.int32) + (a << 12)
    lo = jnp.zeros((_BM, 1), jnp.int32)
    hi = jnp.full((_BM, 1), _N - 1, jnp.int32)
    for _ in range(12):
        mid = (lo + hi) >> 1
        cnt = jnp.sum((mrank <= mid).astype(jnp.float32),
                      axis=1, keepdims=True)
        ge = cnt >= _K
        hi = jnp.where(ge, mid, hi)
        lo = jnp.where(ge, lo, mid + 1)
    sel = jnp.where(mrank <= lo, 1.0, 0.0)
    _finish(s, ls, af, inv_cnt, sel, ones, o_ref)


def kernel(X, A):
    X2 = X[0]
    A2 = A[0].astype(jnp.int32)
    grid = _N // _BM
    blk = lambda i: (i, 0)
    out_spec = pl.BlockSpec((1, 1, 1), lambda i: (0, 0, 0))
    out_shape = jax.ShapeDtypeStruct((1, 1, 1), jnp.float32)

    loss_fast, flags = pl.pallas_call(
        _fast_body,
        grid=(grid,),
        in_specs=[
            pl.BlockSpec((_BM, _D), blk),
            pl.BlockSpec((_N, _D), lambda i: (0, 0)),
            pl.BlockSpec((_BM, _N), blk),
            pl.BlockSpec((_BM, _N), blk),
        ],
        out_specs=(out_spec, out_spec),
        out_shape=(out_shape, out_shape),
    )(X2, X2, A2, jnp.asarray(_RANKS8))

    def full_loss():
        parts = pl.pallas_call(
            _full_body,
            grid=(grid,),
            in_specs=[
                pl.BlockSpec((_BM, _D), blk),
                pl.BlockSpec((_N, _D), lambda i: (0, 0)),
                pl.BlockSpec((_BM, _N), blk),
                pl.BlockSpec((_BM, _N), blk),
            ],
            out_specs=out_spec,
            out_shape=out_shape,
        )(X2, X2, A2, jnp.asarray(_RANKS16))
        return parts[0, 0, 0]

    return lax.cond(flags[0, 0, 0] > 0.0, full_loss,
                    lambda: loss_fast[0, 0, 0])
